# Initial kernel scaffold; baseline (speedup 1.0000x reference)
import jax
import jax.numpy as jnp
from jax.experimental import pallas as pl


def kernel(h_t, edge_index, batch, alpha):
    raise NotImplementedError("write your pallas kernel here")


# speedup vs baseline: 44.9606x; 44.9606x over previous
import functools

import jax
import jax.numpy as jnp
from jax import lax
from jax.experimental import pallas as pl
from jax.experimental.pallas import tpu as pltpu
from jax.experimental.pallas import tpu_sc as plsc

EPS = 0.01
CLAMP_STEP = 0.1
NUM_GRAPHS = 64

NC = 2
NS = 16
NT = NC * NS

RB = 400
CB = 512


def _counts_body(b_ref, k_ref):
  b = b_ref[...]
  g = lax.broadcasted_iota(jnp.int32, (NUM_GRAPHS, 1), 0)
  oh = (b == g).astype(jnp.float32)
  ones = jnp.ones((1, b.shape[1]), jnp.float32)
  counts = lax.dot_general(
      ones, oh, (((1,), (1,)), ((), ())),
      precision=lax.Precision.HIGHEST, preferred_element_type=jnp.float32)
  cnt_node = lax.dot_general(
      counts, oh, (((1,), (0,)), ((), ())),
      precision=lax.Precision.HIGHEST, preferred_element_type=jnp.float32)
  k_ref[...] = lax.rsqrt(jnp.maximum(cnt_node, 1.0))


def _counts_call(bp, np_):
  return pl.pallas_call(
      _counts_body,
      out_shape=jax.ShapeDtypeStruct((1, np_), jnp.float32),
  )(bp)


def _make_attract(np_, ch):
  ept = ch * 128
  mesh = plsc.VectorSubcoreMesh(
      core_axis_name="c", subcore_axis_name="s", num_cores=NC, num_subcores=NS)
  slc = np_ // NS

  @functools.partial(
      pl.kernel,
      out_type=[jax.ShapeDtypeStruct((NC, np_), jnp.float32),
                jax.ShapeDtypeStruct((NC, np_), jnp.float32)],
      mesh=mesh,
      scratch_types=[
          pltpu.VMEM((np_,), jnp.float32),
          pltpu.VMEM((np_,), jnp.float32),
          pltpu.VMEM((np_,), jnp.float32),
          pltpu.VMEM((ch, 128), jnp.int32),
          pltpu.VMEM((ch, 128), jnp.int32),
          pltpu.VMEM((ept,), jnp.float32),
          pltpu.VMEM((ept,), jnp.float32),
          pltpu.VMEM((ept,), jnp.float32),
          pltpu.VMEM((ept,), jnp.float32),
          pltpu.VMEM((slc,), jnp.float32),
          pltpu.VMEM_SHARED((np_,), jnp.float32),
          pltpu.VMEM_SHARED((np_,), jnp.float32),
      ],
      compiler_params=pltpu.CompilerParams(needs_layout_passes=False))
  def attract(x_hbm, y_hbm, k_hbm, rows_hbm, cols_hbm, outx_hbm, outy_hbm,
              x_vm, y_vm, k_vm, ridx, cidx, avx, avy, nvx, nvy, tbuf,
              shx, shy):
    c = lax.axis_index("c")
    s = lax.axis_index("s")
    tid = c * NS + s

    pltpu.sync_copy(x_hbm, x_vm)
    pltpu.sync_copy(y_hbm, y_vm)
    pltpu.sync_copy(k_hbm, k_vm)
    pltpu.sync_copy(rows_hbm.at[tid], ridx)
    pltpu.sync_copy(cols_hbm.at[tid], cidx)

    def zero_body(i, _):
      tbuf[pl.ds(i * 16, 16)] = jnp.zeros((16,), jnp.float32)
      return 0
    lax.fori_loop(0, slc // 16, zero_body, 0)
    pltpu.sync_copy(tbuf, shx.at[pl.ds(s * slc, slc)])
    pltpu.sync_copy(tbuf, shy.at[pl.ds(s * slc, slc)])

    def edge_chunk(q, j):
      ri = ridx[j, pl.ds(q * 16, 16)]
      ci = cidx[j, pl.ds(q * 16, 16)]
      xr = plsc.load_gather(x_vm, [ri])
      xc = plsc.load_gather(x_vm, [ci])
      yr = plsc.load_gather(y_vm, [ri])
      yc = plsc.load_gather(y_vm, [ci])
      kr = plsc.load_gather(k_vm, [ri])
      dx = xr - xc
      dy = yr - yc
      sq = dx * dx + dy * dy
      bi = plsc.bitcast(sq, jnp.int32)
      w = plsc.bitcast(jnp.int32(0x5F3759DF) - (bi >> 1), jnp.float32)
      xh = 0.5 * sq
      w = w * (1.5 - xh * w * w)
      w = w * (1.5 - xh * w * w)
      w = w * (1.5 - xh * w * w)
      dist = sq * w + EPS
      mag = -(dist / kr)
      ax = mag * dx
      ay = mag * dy
      base = (j * 8 + q) * 16
      avx[pl.ds(base, 16)] = ax
      avy[pl.ds(base, 16)] = ay
      nvx[pl.ds(base, 16)] = -ax
      nvy[pl.ds(base, 16)] = -ay
      return j

    def edge_row(j, _):
      lax.fori_loop(0, 8, edge_chunk, j)
      return 0
    lax.fori_loop(0, ch, edge_row, 0)

    plsc.subcore_barrier()

    def scatter_row(j, _):
      pltpu.sync_copy(avx.at[pl.ds(j * 128, 128)], shx.at[ridx.at[j]], add=True)
      pltpu.sync_copy(avy.at[pl.ds(j * 128, 128)], shy.at[ridx.at[j]], add=True)
      pltpu.sync_copy(nvx.at[pl.ds(j * 128, 128)], shx.at[cidx.at[j]], add=True)
      pltpu.sync_copy(nvy.at[pl.ds(j * 128, 128)], shy.at[cidx.at[j]], add=True)
      return 0
    lax.fori_loop(0, ch, scatter_row, 0)

    plsc.subcore_barrier()

    pltpu.sync_copy(shx.at[pl.ds(s * slc, slc)], tbuf)
    pltpu.sync_copy(tbuf, outx_hbm.at[c, pl.ds(s * slc, slc)])
    pltpu.sync_copy(shy.at[pl.ds(s * slc, slc)], tbuf)
    pltpu.sync_copy(tbuf, outy_hbm.at[c, pl.ds(s * slc, slc)])

  return attract


def _repulse_body(xr_ref, yr_ref, br_ref, xc_ref, yc_ref, bc_ref,
                  ox_ref, oy_ref):
  br = br_ref[...]
  bc_full = bc_ref[...]
  b_min = jnp.min(br)
  b_max = jnp.max(br)
  lo = jnp.sum((bc_full < b_min).astype(jnp.int32))
  hi = jnp.sum((bc_full <= b_max).astype(jnp.int32))
  c0 = lo // CB
  c1 = (hi + CB - 1) // CB
  xr = xr_ref[...]
  yr = yr_ref[...]

  def body(ci, carry):
    ax, ay, cnt = carry
    st = ci * CB
    xc = xc_ref[0, pl.ds(st, CB)][None, :]
    yc = yc_ref[0, pl.ds(st, CB)][None, :]
    bcc = bc_ref[0, pl.ds(st, CB)][None, :]
    dx = xr - xc
    dy = yr - yc
    sq = dx * dx + dy * dy
    dist = jnp.sqrt(sq) + EPS
    inv = 1.0 / dist
    m = (bcc == br).astype(jnp.float32)
    w = inv * inv * m
    ax = ax + jnp.sum(dx * w, axis=1, keepdims=True)
    ay = ay + jnp.sum(dy * w, axis=1, keepdims=True)
    cnt = cnt + jnp.sum(m, axis=1, keepdims=True)
    return ax, ay, cnt

  z = jnp.zeros((RB, 1), jnp.float32)
  ax, ay, cnt = lax.fori_loop(c0, c1, body, (z, z, z))
  k2 = 1.0 / jnp.maximum(cnt, 1.0)
  ox_ref[...] = ax * k2
  oy_ref[...] = ay * k2


def _repulse_call(xr, yr, br, xc, yc, bc, n, np_):
  grid = n // RB
  return pl.pallas_call(
      _repulse_body,
      grid=(grid,),
      in_specs=[
          pl.BlockSpec((RB, 1), lambda t: (t, 0)),
          pl.BlockSpec((RB, 1), lambda t: (t, 0)),
          pl.BlockSpec((RB, 1), lambda t: (t, 0)),
          pl.BlockSpec((1, np_), lambda t: (0, 0)),
          pl.BlockSpec((1, np_), lambda t: (0, 0)),
          pl.BlockSpec((1, np_), lambda t: (0, 0)),
      ],
      out_specs=[
          pl.BlockSpec((RB, 1), lambda t: (t, 0)),
          pl.BlockSpec((RB, 1), lambda t: (t, 0)),
      ],
      out_shape=[jax.ShapeDtypeStruct((n, 1), jnp.float32),
                 jax.ShapeDtypeStruct((n, 1), jnp.float32)],
  )(xr, yr, br, xc, yc, bc)


def _combine_body(al_ref, h_ref, fax_ref, fay_ref, rx_ref, ry_ref, o_ref):
  alpha = al_ref[0, 0]
  fx = fax_ref[:, 0:1] + fax_ref[:, 1:2] + rx_ref[...]
  fy = fay_ref[:, 0:1] + fay_ref[:, 1:2] + ry_ref[...]
  dx = alpha * fx
  dy = alpha * fy
  nrm = jnp.sqrt(dx * dx + dy * dy)
  scale = jnp.minimum(CLAMP_STEP / (nrm + 1e-9), 1.0)
  dx = dx * scale
  dy = dy * scale
  h = h_ref[...]
  d = h.shape[1]
  lane = lax.broadcasted_iota(jnp.int32, (RB, d), 1)
  o_ref[...] = h + jnp.where(lane == 0, dx, 0.0) + jnp.where(lane == 1, dy, 0.0)


def _combine_call(alpha2, h_t, fax, fay, rx, ry, n, d):
  grid = n // RB
  return pl.pallas_call(
      _combine_body,
      grid=(grid,),
      in_specs=[
          pl.BlockSpec(memory_space=pltpu.SMEM),
          pl.BlockSpec((RB, d), lambda t: (t, 0)),
          pl.BlockSpec((RB, 2), lambda t: (t, 0)),
          pl.BlockSpec((RB, 2), lambda t: (t, 0)),
          pl.BlockSpec((RB, 1), lambda t: (t, 0)),
          pl.BlockSpec((RB, 1), lambda t: (t, 0)),
      ],
      out_specs=pl.BlockSpec((RB, d), lambda t: (t, 0)),
      out_shape=jax.ShapeDtypeStruct((n, d), jnp.float32),
  )(alpha2, h_t, fax, fay, rx, ry)


def kernel(h_t, edge_index, batch, alpha):
  n, d = h_t.shape
  e = edge_index.shape[1]

  np_ = ((n + 1 + NS * 8 - 1) // (NS * 8)) * (NS * 8)
  np_ = ((np_ + CB - 1) // CB) * CB
  ept = e // NT
  ch = (ept + 127) // 128
  ept_p = ch * 128

  x = h_t[:, 0]
  y = h_t[:, 1]
  xp = jnp.pad(x, (0, np_ - n))
  yp = jnp.pad(y, (0, np_ - n))
  bp_cnt = jnp.pad(batch, (0, np_ - n), constant_values=NUM_GRAPHS)
  bp_rep = jnp.pad(batch, (0, np_ - n), constant_values=jnp.int32(1 << 30))

  k_row = _counts_call(bp_cnt.reshape(1, np_), np_)
  kp = k_row.reshape(np_)

  rows = edge_index[0].reshape(NT, ept)
  cols = edge_index[1].reshape(NT, ept)
  rows3 = jnp.pad(rows, ((0, 0), (0, ept_p - ept)),
                  constant_values=n).reshape(NT, ch, 128)
  cols3 = jnp.pad(cols, ((0, 0), (0, ept_p - ept)),
                  constant_values=n).reshape(NT, ch, 128)

  attract = _make_attract(np_, ch)
  outx, outy = attract(xp, yp, kp, rows3, cols3)

  fax = outx.T[:n]
  fay = outy.T[:n]

  rx, ry = _repulse_call(
      x[:, None], y[:, None], batch[:, None],
      xp[None, :], yp[None, :], bp_rep[None, :], n, np_)

  alpha2 = jnp.asarray(alpha, jnp.float32).reshape(1, 1)
  return _combine_call(alpha2, h_t, fax, fay, rx, ry, n, d)


# speedup vs baseline: 47.2497x; 1.0509x over previous
import functools

import jax
import jax.numpy as jnp
from jax import lax
from jax.experimental import pallas as pl
from jax.experimental.pallas import tpu as pltpu
from jax.experimental.pallas import tpu_sc as plsc

EPS = 0.01
CLAMP_STEP = 0.1
NUM_GRAPHS = 64

NC = 2
NS = 16
NT = NC * NS

RB = 400
CB = 512


def _counts_body(b_ref, k_ref):
  b = b_ref[...]
  g = lax.broadcasted_iota(jnp.int32, (NUM_GRAPHS, 1), 0)
  oh = (b == g).astype(jnp.float32)
  ones = jnp.ones((1, b.shape[1]), jnp.float32)
  counts = lax.dot_general(
      ones, oh, (((1,), (1,)), ((), ())),
      precision=lax.Precision.HIGHEST, preferred_element_type=jnp.float32)
  cnt_node = lax.dot_general(
      counts, oh, (((1,), (0,)), ((), ())),
      precision=lax.Precision.HIGHEST, preferred_element_type=jnp.float32)
  k_ref[...] = lax.rsqrt(jnp.maximum(cnt_node, 1.0))


def _counts_call(bp, np_):
  return pl.pallas_call(
      _counts_body,
      out_shape=jax.ShapeDtypeStruct((1, np_), jnp.float32),
  )(bp)


def _make_attract(np_, ch):
  ept = ch * 128
  mesh = plsc.VectorSubcoreMesh(
      core_axis_name="c", subcore_axis_name="s", num_cores=NC, num_subcores=NS)
  slc = np_ // NS

  @functools.partial(
      pl.kernel,
      out_type=[jax.ShapeDtypeStruct((NC, np_), jnp.float32),
                jax.ShapeDtypeStruct((NC, np_), jnp.float32)],
      mesh=mesh,
      scratch_types=[
          pltpu.VMEM((np_,), jnp.float32),
          pltpu.VMEM((np_,), jnp.float32),
          pltpu.VMEM((np_,), jnp.float32),
          pltpu.VMEM((ch, 128), jnp.int32),
          pltpu.VMEM((ch, 128), jnp.int32),
          pltpu.VMEM((ept,), jnp.float32),
          pltpu.VMEM((ept,), jnp.float32),
          pltpu.VMEM((ept,), jnp.float32),
          pltpu.VMEM((ept,), jnp.float32),
          pltpu.VMEM((slc,), jnp.float32),
          pltpu.VMEM_SHARED((np_,), jnp.float32),
          pltpu.VMEM_SHARED((np_,), jnp.float32),
          pltpu.SemaphoreType.DMA,
      ],
      compiler_params=pltpu.CompilerParams(needs_layout_passes=False))
  def attract(x_hbm, y_hbm, k_hbm, rows_hbm, cols_hbm, outx_hbm, outy_hbm,
              x_vm, y_vm, k_vm, ridx, cidx, avx, avy, nvx, nvy, tbuf,
              shx, shy, sem):
    c = lax.axis_index("c")
    s = lax.axis_index("s")
    tid = c * NS + s

    pltpu.sync_copy(x_hbm, x_vm)
    pltpu.sync_copy(y_hbm, y_vm)
    pltpu.sync_copy(k_hbm, k_vm)
    pltpu.sync_copy(rows_hbm.at[tid], ridx)
    pltpu.sync_copy(cols_hbm.at[tid], cidx)

    def zero_body(i, _):
      tbuf[pl.ds(i * 16, 16)] = jnp.zeros((16,), jnp.float32)
      return 0
    lax.fori_loop(0, slc // 16, zero_body, 0)
    pltpu.sync_copy(tbuf, shx.at[pl.ds(s * slc, slc)])
    pltpu.sync_copy(tbuf, shy.at[pl.ds(s * slc, slc)])
    plsc.subcore_barrier()

    def edge_chunk(q, j):
      ri = ridx[j, pl.ds(q * 16, 16)]
      ci = cidx[j, pl.ds(q * 16, 16)]
      xr = plsc.load_gather(x_vm, [ri])
      xc = plsc.load_gather(x_vm, [ci])
      yr = plsc.load_gather(y_vm, [ri])
      yc = plsc.load_gather(y_vm, [ci])
      kr = plsc.load_gather(k_vm, [ri])
      dx = xr - xc
      dy = yr - yc
      sq = dx * dx + dy * dy
      bi = plsc.bitcast(sq, jnp.int32)
      w = plsc.bitcast(jnp.int32(0x5F3759DF) - (bi >> 1), jnp.float32)
      xh = 0.5 * sq
      w = w * (1.5 - xh * w * w)
      w = w * (1.5 - xh * w * w)
      dist = sq * w + EPS
      mag = -(dist / kr)
      ax = mag * dx
      ay = mag * dy
      base = (j * 8 + q) * 16
      avx[pl.ds(base, 16)] = ax
      avy[pl.ds(base, 16)] = ay
      nvx[pl.ds(base, 16)] = -ax
      nvy[pl.ds(base, 16)] = -ay
      return j

    W = 4

    def fire(j):
      pltpu.async_copy(avx.at[pl.ds(j * 128, 128)], shx.at[ridx.at[j]],
                       sem, add=True)
      pltpu.async_copy(avy.at[pl.ds(j * 128, 128)], shy.at[ridx.at[j]],
                       sem, add=True)
      pltpu.async_copy(nvx.at[pl.ds(j * 128, 128)], shx.at[cidx.at[j]],
                       sem, add=True)
      pltpu.async_copy(nvy.at[pl.ds(j * 128, 128)], shy.at[cidx.at[j]],
                       sem, add=True)

    def drain(j):
      pltpu.make_async_copy(avx.at[pl.ds(j * 128, 128)], shx.at[ridx.at[j]],
                            sem).wait()
      pltpu.make_async_copy(avy.at[pl.ds(j * 128, 128)], shy.at[ridx.at[j]],
                            sem).wait()
      pltpu.make_async_copy(nvx.at[pl.ds(j * 128, 128)], shx.at[cidx.at[j]],
                            sem).wait()
      pltpu.make_async_copy(nvy.at[pl.ds(j * 128, 128)], shy.at[cidx.at[j]],
                            sem).wait()

    def edge_row(j, _):
      lax.fori_loop(0, 8, edge_chunk, j)
      fire(j)

      @pl.when(j >= W)
      def _():
        drain(j - W)
      return 0
    lax.fori_loop(0, ch, edge_row, 0)

    def drain_tail(i, _):
      drain(ch - W + i)
      return 0
    lax.fori_loop(0, W, drain_tail, 0)

    plsc.subcore_barrier()

    pltpu.sync_copy(shx.at[pl.ds(s * slc, slc)], tbuf)
    pltpu.sync_copy(tbuf, outx_hbm.at[c, pl.ds(s * slc, slc)])
    pltpu.sync_copy(shy.at[pl.ds(s * slc, slc)], tbuf)
    pltpu.sync_copy(tbuf, outy_hbm.at[c, pl.ds(s * slc, slc)])

  return attract


def _repulse_body(xr_ref, yr_ref, br_ref, xc_ref, yc_ref, bc_ref,
                  ox_ref, oy_ref):
  br = br_ref[...]
  bc_full = bc_ref[...]
  b_min = jnp.min(br)
  b_max = jnp.max(br)
  lo = jnp.sum((bc_full < b_min).astype(jnp.int32))
  hi = jnp.sum((bc_full <= b_max).astype(jnp.int32))
  c0 = lo // CB
  c1 = (hi + CB - 1) // CB
  xr = xr_ref[...]
  yr = yr_ref[...]

  def body(ci, carry):
    ax, ay, cnt = carry
    st = ci * CB
    xc = xc_ref[0, pl.ds(st, CB)][None, :]
    yc = yc_ref[0, pl.ds(st, CB)][None, :]
    bcc = bc_ref[0, pl.ds(st, CB)][None, :]
    dx = xr - xc
    dy = yr - yc
    sq = dx * dx + dy * dy
    dist = jnp.sqrt(sq) + EPS
    inv = 1.0 / dist
    m = (bcc == br).astype(jnp.float32)
    w = inv * inv * m
    ax = ax + jnp.sum(dx * w, axis=1, keepdims=True)
    ay = ay + jnp.sum(dy * w, axis=1, keepdims=True)
    cnt = cnt + jnp.sum(m, axis=1, keepdims=True)
    return ax, ay, cnt

  z = jnp.zeros((RB, 1), jnp.float32)
  ax, ay, cnt = lax.fori_loop(c0, c1, body, (z, z, z))
  k2 = 1.0 / jnp.maximum(cnt, 1.0)
  ox_ref[...] = ax * k2
  oy_ref[...] = ay * k2


def _repulse_call(xr, yr, br, xc, yc, bc, n, np_):
  grid = n // RB
  return pl.pallas_call(
      _repulse_body,
      grid=(grid,),
      in_specs=[
          pl.BlockSpec((RB, 1), lambda t: (t, 0)),
          pl.BlockSpec((RB, 1), lambda t: (t, 0)),
          pl.BlockSpec((RB, 1), lambda t: (t, 0)),
          pl.BlockSpec((1, np_), lambda t: (0, 0)),
          pl.BlockSpec((1, np_), lambda t: (0, 0)),
          pl.BlockSpec((1, np_), lambda t: (0, 0)),
      ],
      out_specs=[
          pl.BlockSpec((RB, 1), lambda t: (t, 0)),
          pl.BlockSpec((RB, 1), lambda t: (t, 0)),
      ],
      out_shape=[jax.ShapeDtypeStruct((n, 1), jnp.float32),
                 jax.ShapeDtypeStruct((n, 1), jnp.float32)],
  )(xr, yr, br, xc, yc, bc)


def _combine_body(al_ref, h_ref, fax_ref, fay_ref, rx_ref, ry_ref, o_ref):
  alpha = al_ref[0, 0]
  fx = fax_ref[:, 0:1] + fax_ref[:, 1:2] + rx_ref[...]
  fy = fay_ref[:, 0:1] + fay_ref[:, 1:2] + ry_ref[...]
  dx = alpha * fx
  dy = alpha * fy
  nrm = jnp.sqrt(dx * dx + dy * dy)
  scale = jnp.minimum(CLAMP_STEP / (nrm + 1e-9), 1.0)
  dx = dx * scale
  dy = dy * scale
  h = h_ref[...]
  d = h.shape[1]
  lane = lax.broadcasted_iota(jnp.int32, (RB, d), 1)
  o_ref[...] = h + jnp.where(lane == 0, dx, 0.0) + jnp.where(lane == 1, dy, 0.0)


def _combine_call(alpha2, h_t, fax, fay, rx, ry, n, d):
  grid = n // RB
  return pl.pallas_call(
      _combine_body,
      grid=(grid,),
      in_specs=[
          pl.BlockSpec(memory_space=pltpu.SMEM),
          pl.BlockSpec((RB, d), lambda t: (t, 0)),
          pl.BlockSpec((RB, 2), lambda t: (t, 0)),
          pl.BlockSpec((RB, 2), lambda t: (t, 0)),
          pl.BlockSpec((RB, 1), lambda t: (t, 0)),
          pl.BlockSpec((RB, 1), lambda t: (t, 0)),
      ],
      out_specs=pl.BlockSpec((RB, d), lambda t: (t, 0)),
      out_shape=jax.ShapeDtypeStruct((n, d), jnp.float32),
  )(alpha2, h_t, fax, fay, rx, ry)


def kernel(h_t, edge_index, batch, alpha):
  n, d = h_t.shape
  e = edge_index.shape[1]

  np_ = ((n + 1 + NS * 8 - 1) // (NS * 8)) * (NS * 8)
  np_ = ((np_ + CB - 1) // CB) * CB
  ept = e // NT
  ch = (ept + 127) // 128
  ept_p = ch * 128

  x = h_t[:, 0]
  y = h_t[:, 1]
  xp = jnp.pad(x, (0, np_ - n))
  yp = jnp.pad(y, (0, np_ - n))
  bp_cnt = jnp.pad(batch, (0, np_ - n), constant_values=NUM_GRAPHS)
  bp_rep = jnp.pad(batch, (0, np_ - n), constant_values=jnp.int32(1 << 30))

  k_row = _counts_call(bp_cnt.reshape(1, np_), np_)
  kp = k_row.reshape(np_)

  rows = edge_index[0].reshape(NT, ept)
  cols = edge_index[1].reshape(NT, ept)
  rows3 = jnp.pad(rows, ((0, 0), (0, ept_p - ept)),
                  constant_values=n).reshape(NT, ch, 128)
  cols3 = jnp.pad(cols, ((0, 0), (0, ept_p - ept)),
                  constant_values=n).reshape(NT, ch, 128)

  attract = _make_attract(np_, ch)
  outx, outy = attract(xp, yp, kp, rows3, cols3)

  fax = outx.T[:n]
  fay = outy.T[:n]

  rx, ry = _repulse_call(
      x[:, None], y[:, None], batch[:, None],
      xp[None, :], yp[None, :], bp_rep[None, :], n, np_)

  alpha2 = jnp.asarray(alpha, jnp.float32).reshape(1, 1)
  return _combine_call(alpha2, h_t, fax, fay, rx, ry, n, d)


# speedup vs baseline: 47.2900x; 1.0009x over previous
import functools

import jax
import jax.numpy as jnp
from jax import lax
from jax.experimental import pallas as pl
from jax.experimental.pallas import tpu as pltpu
from jax.experimental.pallas import tpu_sc as plsc

EPS = 0.01
CLAMP_STEP = 0.1
NUM_GRAPHS = 64

NC = 2
NS = 16
NT = NC * NS

RB = 400
CB = 512


def _counts_body(b_ref, k_ref):
  b = b_ref[...]
  g = lax.broadcasted_iota(jnp.int32, (NUM_GRAPHS, 1), 0)
  oh = (b == g).astype(jnp.float32)
  ones = jnp.ones((1, b.shape[1]), jnp.float32)
  counts = lax.dot_general(
      ones, oh, (((1,), (1,)), ((), ())),
      precision=lax.Precision.HIGHEST, preferred_element_type=jnp.float32)
  cnt_node = lax.dot_general(
      counts, oh, (((1,), (0,)), ((), ())),
      precision=lax.Precision.HIGHEST, preferred_element_type=jnp.float32)
  k_ref[...] = lax.rsqrt(jnp.maximum(cnt_node, 1.0))


def _counts_call(bp, np_):
  return pl.pallas_call(
      _counts_body,
      out_shape=jax.ShapeDtypeStruct((1, np_), jnp.float32),
  )(bp)


def _make_attract(np_, ch):
  ept = ch * 128
  mesh = plsc.VectorSubcoreMesh(
      core_axis_name="c", subcore_axis_name="s", num_cores=NC, num_subcores=NS)
  slc = np_ // NS

  @functools.partial(
      pl.kernel,
      out_type=[jax.ShapeDtypeStruct((NC, np_), jnp.float32),
                jax.ShapeDtypeStruct((NC, np_), jnp.float32)],
      mesh=mesh,
      scratch_types=[
          pltpu.VMEM((np_,), jnp.float32),
          pltpu.VMEM((np_,), jnp.float32),
          pltpu.VMEM((np_,), jnp.float32),
          pltpu.VMEM((ch, 128), jnp.int32),
          pltpu.VMEM((ch, 128), jnp.int32),
          pltpu.VMEM((ept,), jnp.float32),
          pltpu.VMEM((ept,), jnp.float32),
          pltpu.VMEM((ept,), jnp.float32),
          pltpu.VMEM((ept,), jnp.float32),
          pltpu.VMEM((slc,), jnp.float32),
          pltpu.VMEM_SHARED((np_,), jnp.float32),
          pltpu.VMEM_SHARED((np_,), jnp.float32),
          pltpu.SemaphoreType.DMA,
      ],
      compiler_params=pltpu.CompilerParams(needs_layout_passes=False))
  def attract(x_hbm, y_hbm, k_hbm, rows_hbm, cols_hbm, outx_hbm, outy_hbm,
              x_vm, y_vm, k_vm, ridx, cidx, avx, avy, nvx, nvy, tbuf,
              shx, shy, sem):
    c = lax.axis_index("c")
    s = lax.axis_index("s")
    tid = c * NS + s

    pltpu.sync_copy(x_hbm, x_vm)
    pltpu.sync_copy(y_hbm, y_vm)
    pltpu.sync_copy(k_hbm, k_vm)
    pltpu.sync_copy(rows_hbm.at[tid], ridx)
    pltpu.sync_copy(cols_hbm.at[tid], cidx)

    def zero_body(i, _):
      tbuf[pl.ds(i * 16, 16)] = jnp.zeros((16,), jnp.float32)
      return 0
    lax.fori_loop(0, slc // 16, zero_body, 0)
    pltpu.sync_copy(tbuf, shx.at[pl.ds(s * slc, slc)])
    pltpu.sync_copy(tbuf, shy.at[pl.ds(s * slc, slc)])
    plsc.subcore_barrier()

    def edge_chunk(q, j):
      ri = ridx[j, pl.ds(q * 16, 16)]
      ci = cidx[j, pl.ds(q * 16, 16)]
      xr = plsc.load_gather(x_vm, [ri])
      xc = plsc.load_gather(x_vm, [ci])
      yr = plsc.load_gather(y_vm, [ri])
      yc = plsc.load_gather(y_vm, [ci])
      kr = plsc.load_gather(k_vm, [ri])
      dx = xr - xc
      dy = yr - yc
      sq = dx * dx + dy * dy
      bi = plsc.bitcast(sq, jnp.int32)
      w = plsc.bitcast(jnp.int32(0x5F3759DF) - (bi >> 1), jnp.float32)
      xh = 0.5 * sq
      w = w * (1.5 - xh * w * w)
      w = w * (1.5 - xh * w * w)
      dist = sq * w + EPS
      mag = -(dist / kr)
      ax = mag * dx
      ay = mag * dy
      base = (j * 8 + q) * 16
      avx[pl.ds(base, 16)] = ax
      avy[pl.ds(base, 16)] = ay
      nvx[pl.ds(base, 16)] = -ax
      nvy[pl.ds(base, 16)] = -ay
      return j

    W = 4

    def fire(j):
      pltpu.async_copy(avx.at[pl.ds(j * 128, 128)], shx.at[ridx.at[j]],
                       sem, add=True)
      pltpu.async_copy(avy.at[pl.ds(j * 128, 128)], shy.at[ridx.at[j]],
                       sem, add=True)
      pltpu.async_copy(nvx.at[pl.ds(j * 128, 128)], shx.at[cidx.at[j]],
                       sem, add=True)
      pltpu.async_copy(nvy.at[pl.ds(j * 128, 128)], shy.at[cidx.at[j]],
                       sem, add=True)

    def drain(j):
      pltpu.make_async_copy(avx.at[pl.ds(j * 128, 128)], shx.at[ridx.at[j]],
                            sem).wait()
      pltpu.make_async_copy(avy.at[pl.ds(j * 128, 128)], shy.at[ridx.at[j]],
                            sem).wait()
      pltpu.make_async_copy(nvx.at[pl.ds(j * 128, 128)], shx.at[cidx.at[j]],
                            sem).wait()
      pltpu.make_async_copy(nvy.at[pl.ds(j * 128, 128)], shy.at[cidx.at[j]],
                            sem).wait()

    def edge_row(j, _):
      for q in range(8):
        edge_chunk(q, j)
      fire(j)

      @pl.when(j >= W)
      def _():
        drain(j - W)
      return 0
    lax.fori_loop(0, ch, edge_row, 0)

    def drain_tail(i, _):
      drain(ch - W + i)
      return 0
    lax.fori_loop(0, W, drain_tail, 0)

    plsc.subcore_barrier()

    pltpu.sync_copy(shx.at[pl.ds(s * slc, slc)], tbuf)
    pltpu.sync_copy(tbuf, outx_hbm.at[c, pl.ds(s * slc, slc)])
    pltpu.sync_copy(shy.at[pl.ds(s * slc, slc)], tbuf)
    pltpu.sync_copy(tbuf, outy_hbm.at[c, pl.ds(s * slc, slc)])

  return attract


def _repulse_body(xr_ref, yr_ref, br_ref, xc_ref, yc_ref, bc_ref,
                  ox_ref, oy_ref):
  br = br_ref[...]
  bc_full = bc_ref[...]
  b_min = jnp.min(br)
  b_max = jnp.max(br)
  lo = jnp.sum((bc_full < b_min).astype(jnp.int32))
  hi = jnp.sum((bc_full <= b_max).astype(jnp.int32))
  c0 = lo // CB
  c1 = (hi + CB - 1) // CB
  xr = xr_ref[...]
  yr = yr_ref[...]

  def body(ci, carry):
    ax, ay, cnt = carry
    st = ci * CB
    xc = xc_ref[0, pl.ds(st, CB)][None, :]
    yc = yc_ref[0, pl.ds(st, CB)][None, :]
    bcc = bc_ref[0, pl.ds(st, CB)][None, :]
    dx = xr - xc
    dy = yr - yc
    sq = dx * dx + dy * dy
    dist = jnp.sqrt(sq) + EPS
    inv = 1.0 / dist
    m = (bcc == br).astype(jnp.float32)
    w = inv * inv * m
    ax = ax + jnp.sum(dx * w, axis=1, keepdims=True)
    ay = ay + jnp.sum(dy * w, axis=1, keepdims=True)
    cnt = cnt + jnp.sum(m, axis=1, keepdims=True)
    return ax, ay, cnt

  z = jnp.zeros((RB, 1), jnp.float32)
  ax, ay, cnt = lax.fori_loop(c0, c1, body, (z, z, z))
  k2 = 1.0 / jnp.maximum(cnt, 1.0)
  ox_ref[...] = ax * k2
  oy_ref[...] = ay * k2


def _repulse_call(xr, yr, br, xc, yc, bc, n, np_):
  grid = n // RB
  return pl.pallas_call(
      _repulse_body,
      grid=(grid,),
      in_specs=[
          pl.BlockSpec((RB, 1), lambda t: (t, 0)),
          pl.BlockSpec((RB, 1), lambda t: (t, 0)),
          pl.BlockSpec((RB, 1), lambda t: (t, 0)),
          pl.BlockSpec((1, np_), lambda t: (0, 0)),
          pl.BlockSpec((1, np_), lambda t: (0, 0)),
          pl.BlockSpec((1, np_), lambda t: (0, 0)),
      ],
      out_specs=[
          pl.BlockSpec((RB, 1), lambda t: (t, 0)),
          pl.BlockSpec((RB, 1), lambda t: (t, 0)),
      ],
      out_shape=[jax.ShapeDtypeStruct((n, 1), jnp.float32),
                 jax.ShapeDtypeStruct((n, 1), jnp.float32)],
  )(xr, yr, br, xc, yc, bc)


def _combine_body(al_ref, h_ref, fax_ref, fay_ref, rx_ref, ry_ref, o_ref):
  alpha = al_ref[0, 0]
  fx = fax_ref[:, 0:1] + fax_ref[:, 1:2] + rx_ref[...]
  fy = fay_ref[:, 0:1] + fay_ref[:, 1:2] + ry_ref[...]
  dx = alpha * fx
  dy = alpha * fy
  nrm = jnp.sqrt(dx * dx + dy * dy)
  scale = jnp.minimum(CLAMP_STEP / (nrm + 1e-9), 1.0)
  dx = dx * scale
  dy = dy * scale
  h = h_ref[...]
  d = h.shape[1]
  lane = lax.broadcasted_iota(jnp.int32, (RB, d), 1)
  o_ref[...] = h + jnp.where(lane == 0, dx, 0.0) + jnp.where(lane == 1, dy, 0.0)


def _combine_call(alpha2, h_t, fax, fay, rx, ry, n, d):
  grid = n // RB
  return pl.pallas_call(
      _combine_body,
      grid=(grid,),
      in_specs=[
          pl.BlockSpec(memory_space=pltpu.SMEM),
          pl.BlockSpec((RB, d), lambda t: (t, 0)),
          pl.BlockSpec((RB, 2), lambda t: (t, 0)),
          pl.BlockSpec((RB, 2), lambda t: (t, 0)),
          pl.BlockSpec((RB, 1), lambda t: (t, 0)),
          pl.BlockSpec((RB, 1), lambda t: (t, 0)),
      ],
      out_specs=pl.BlockSpec((RB, d), lambda t: (t, 0)),
      out_shape=jax.ShapeDtypeStruct((n, d), jnp.float32),
  )(alpha2, h_t, fax, fay, rx, ry)


def kernel(h_t, edge_index, batch, alpha):
  n, d = h_t.shape
  e = edge_index.shape[1]

  np_ = ((n + 1 + NS * 8 - 1) // (NS * 8)) * (NS * 8)
  np_ = ((np_ + CB - 1) // CB) * CB
  ept = e // NT
  ch = (ept + 127) // 128
  ept_p = ch * 128

  x = h_t[:, 0]
  y = h_t[:, 1]
  xp = jnp.pad(x, (0, np_ - n))
  yp = jnp.pad(y, (0, np_ - n))
  bp_cnt = jnp.pad(batch, (0, np_ - n), constant_values=NUM_GRAPHS)
  bp_rep = jnp.pad(batch, (0, np_ - n), constant_values=jnp.int32(1 << 30))

  k_row = _counts_call(bp_cnt.reshape(1, np_), np_)
  kp = k_row.reshape(np_)

  rows = edge_index[0].reshape(NT, ept)
  cols = edge_index[1].reshape(NT, ept)
  rows3 = jnp.pad(rows, ((0, 0), (0, ept_p - ept)),
                  constant_values=n).reshape(NT, ch, 128)
  cols3 = jnp.pad(cols, ((0, 0), (0, ept_p - ept)),
                  constant_values=n).reshape(NT, ch, 128)

  attract = _make_attract(np_, ch)
  outx, outy = attract(xp, yp, kp, rows3, cols3)

  fax = outx.T[:n]
  fay = outy.T[:n]

  rx, ry = _repulse_call(
      x[:, None], y[:, None], batch[:, None],
      xp[None, :], yp[None, :], bp_rep[None, :], n, np_)

  alpha2 = jnp.asarray(alpha, jnp.float32).reshape(1, 1)
  return _combine_call(alpha2, h_t, fax, fay, rx, ry, n, d)
